# Initial kernel scaffold; baseline (speedup 1.0000x reference)
import jax
import jax.numpy as jnp
from jax.experimental import pallas as pl


def kernel(odm_locs, odm_scores, boxes, labels, priors_cxcy):
    raise NotImplementedError("write your pallas kernel here")


# speedup vs baseline: 43.2085x; 43.2085x over previous
import functools
from math import sqrt

import jax
import jax.numpy as jnp
from jax import lax
from jax.experimental import pallas as pl
from jax.experimental.pallas import tpu as pltpu

_N_CLASSES = 2
_THRESHOLD = 0.5
_NEG_POS_RATIO = 3
_ALPHA = 1.0
_SMOOTHING = 0.05
_N_BISECT = 34


def _loss_kernel(locs_ref, scores_ref, boxes_ref, labels_ref, priors_ref,
                 out_ref, lab_s, tx1_s, ty1_s, tx2_s, ty2_s, *, B, P, Pp):
    b = pl.program_id(0)
    f32 = jnp.float32

    pcx = priors_ref[0:1, :]
    pcy = priors_ref[1:2, :]
    pw = priors_ref[2:3, :]
    ph = priors_ref[3:4, :]
    px1 = pcx - pw * 0.5
    py1 = pcy - ph * 0.5
    px2 = pcx + pw * 0.5
    py2 = pcy + ph * 0.5

    col1 = lax.broadcasted_iota(jnp.int32, (1, Pp), 1)
    valid_col = col1 < P

    bx = boxes_ref[b]
    x1 = bx[:, 0:1]
    y1 = bx[:, 1:2]
    x2 = bx[:, 2:3]
    y2 = bx[:, 3:4]
    lab_b = labels_ref[b]

    ix1 = jnp.maximum(x1, px1)
    iy1 = jnp.maximum(y1, py1)
    ix2 = jnp.minimum(x2, px2)
    iy2 = jnp.minimum(y2, py2)
    inter = jnp.maximum(ix2 - ix1, 0.0) * jnp.maximum(iy2 - iy1, 0.0)
    a1 = (x2 - x1) * (y2 - y1)
    a2 = (px2 - px1) * (py2 - py1)
    ov = inter / (a1 + a2 - inter + 1e-10)
    ov = jnp.where(valid_col, ov, -1.0)

    eio = lax.broadcasted_iota(jnp.int32, (8, Pp), 0)
    cio = lax.broadcasted_iota(jnp.int32, (8, Pp), 1)

    ofp = jnp.max(ov, axis=0, keepdims=True)
    oep = jnp.min(jnp.where(ov == ofp, eio, 8), axis=0, keepdims=True)
    ofe = jnp.max(ov, axis=1, keepdims=True)
    pfe = jnp.min(jnp.where(ov == ofe, cio, Pp), axis=1, keepdims=True)

    match = cio == pfe
    validk = ofe > 0.0
    e_last_all = jnp.max(jnp.where(match, eio, -1), axis=0, keepdims=True)
    e_last_val = jnp.max(jnp.where(match & validk, eio, -1), axis=0,
                         keepdims=True)
    force = (e_last_all >= 0) & (e_last_all == e_last_val)
    ofp = jnp.where(force, 1.0, ofp)
    oep = jnp.where(force, e_last_all, oep)

    onehot = oep == eio
    label_fp = jnp.sum(jnp.where(onehot, lab_b, 0.0), axis=0, keepdims=True)
    tx1 = jnp.sum(jnp.where(onehot, x1, 0.0), axis=0, keepdims=True)
    ty1 = jnp.sum(jnp.where(onehot, y1, 0.0), axis=0, keepdims=True)
    tx2 = jnp.sum(jnp.where(onehot, x2, 0.0), axis=0, keepdims=True)
    ty2 = jnp.sum(jnp.where(onehot, y2, 0.0), axis=0, keepdims=True)
    label_fp = jnp.where(ofp < _THRESHOLD - 0.1, 0.0, label_fp)

    lab_s[pl.ds(b, 1), :] = label_fp
    tx1_s[pl.ds(b, 1), :] = tx1
    ty1_s[pl.ds(b, 1), :] = ty1
    tx2_s[pl.ds(b, 1), :] = tx2
    ty2_s[pl.ds(b, 1), :] = ty2

    @pl.when(b == B - 1)
    def _dense():
        lab = lab_s[...]
        pos = lab > 0.0
        posf = pos.astype(f32)
        n_pos_vec = jnp.sum(posf, axis=1, keepdims=True)
        n_pos_total = jnp.sum(posf)

        gcx = locs_ref[0]
        gcy = locs_ref[1]
        gw = locs_ref[2]
        gh = locs_ref[3]
        cx = gcx * pw * 0.1 + pcx
        cy = gcy * ph * 0.1 + pcy
        w = jnp.exp(gw * 0.2) * pw
        h = jnp.exp(gh * 0.2) * ph
        dx1 = cx - w * 0.5
        dy1 = cy - h * 0.5
        dx2 = cx + w * 0.5
        dy2 = cy + h * 0.5

        ttx1 = tx1_s[...]
        tty1 = ty1_s[...]
        ttx2 = tx2_s[...]
        tty2 = ty2_s[...]

        lx1 = jnp.maximum(dx1, ttx1)
        ly1 = jnp.maximum(dy1, tty1)
        lx2 = jnp.minimum(dx2, ttx2)
        ly2 = jnp.minimum(dy2, tty2)
        inter_d = (jnp.maximum(lx2 - lx1, 0.0) * jnp.maximum(ly2 - ly1, 0.0))
        ap = jnp.maximum(dx2 - dx1, 0.0) * jnp.maximum(dy2 - dy1, 0.0)
        at = (ttx2 - ttx1) * (tty2 - tty1)
        iou = inter_d / (ap + at - inter_d + 1e-7)
        dcx = (dx1 + dx2) - (ttx1 + ttx2)
        dcy = (dy1 + dy2) - (tty1 + tty2)
        d2 = (dcx * dcx + dcy * dcy) * 0.25
        ex1 = jnp.minimum(dx1, ttx1)
        ey1 = jnp.minimum(dy1, tty1)
        ex2 = jnp.maximum(dx2, ttx2)
        ey2 = jnp.maximum(dy2, tty2)
        c2 = (ex2 - ex1) ** 2 + (ey2 - ey1) ** 2 + 1e-7
        per_box = 1.0 - iou + d2 / c2
        loc_sum = jnp.sum(jnp.where(pos, per_box, 0.0))

        s0 = scores_ref[0]
        s1 = scores_ref[1]
        m = jnp.maximum(s0, s1)
        lse = m + jnp.log(jnp.exp(s0 - m) + jnp.exp(s1 - m))
        lp0 = s0 - lse
        lp1 = s1 - lse
        lp_t = jnp.where(lab > 0.0, lp1, lp0)
        eps_i = _SMOOTHING / (_N_CLASSES - 1)
        ce = -((1.0 - _SMOOTHING) * lp_t + eps_i * (lp0 + lp1 - lp_t))
        conf_pos_sum = jnp.sum(jnp.where(pos, ce, 0.0))
        cn = jnp.where(valid_col & ~pos, ce, 0.0)

        k = jnp.minimum(_NEG_POS_RATIO * n_pos_vec, float(P))
        hi0 = jnp.max(cn, axis=1, keepdims=True) + 1.0
        lo0 = jnp.zeros_like(hi0)

        def body(_, carry):
            lo, hi = carry
            mid = 0.5 * (lo + hi)
            cnt = jnp.sum(jnp.where(cn > mid, 1.0, 0.0), axis=1,
                          keepdims=True)
            gt = cnt > k
            return jnp.where(gt, mid, lo), jnp.where(gt, hi, mid)

        _, hi = lax.fori_loop(0, _N_BISECT, body, (lo0, hi0))
        above = cn > hi
        cnt_hi = jnp.sum(jnp.where(above, 1.0, 0.0), axis=1, keepdims=True)
        sum_hi = jnp.sum(jnp.where(above, cn, 0.0), axis=1, keepdims=True)
        hard_sum = jnp.sum(sum_hi + (k - cnt_hi) * hi)

        conf_loss = (hard_sum + conf_pos_sum) / n_pos_total
        loc_loss = loc_sum / jnp.maximum(n_pos_total, 1.0)
        total = conf_loss + _ALPHA * loc_loss
        out_ref[...] = jnp.broadcast_to(total, (1, 1))


@jax.jit
def kernel(odm_locs, odm_scores, boxes, labels, priors_cxcy):
    B, P, C = odm_scores.shape
    Pp = ((P + 127) // 128) * 128
    pad = Pp - P
    locs4 = jnp.pad(jnp.transpose(odm_locs, (2, 0, 1)),
                    ((0, 0), (0, 0), (0, pad)))
    scores2 = jnp.pad(jnp.transpose(odm_scores, (2, 0, 1)),
                      ((0, 0), (0, 0), (0, pad)))
    priors_t = jnp.pad(priors_cxcy.T, ((0, 0), (0, pad)))
    labels_f = labels.astype(jnp.float32)[..., None]

    body = functools.partial(_loss_kernel, B=B, P=P, Pp=Pp)
    out = pl.pallas_call(
        body,
        grid=(B,),
        in_specs=[
            pl.BlockSpec((4, B, Pp), lambda b: (0, 0, 0)),
            pl.BlockSpec((C, B, Pp), lambda b: (0, 0, 0)),
            pl.BlockSpec(boxes.shape, lambda b: (0, 0, 0)),
            pl.BlockSpec(labels_f.shape, lambda b: (0, 0, 0)),
            pl.BlockSpec((4, Pp), lambda b: (0, 0)),
        ],
        out_specs=pl.BlockSpec((1, 1), lambda b: (0, 0)),
        out_shape=jax.ShapeDtypeStruct((1, 1), jnp.float32),
        scratch_shapes=[pltpu.VMEM((B, Pp), jnp.float32) for _ in range(5)],
        compiler_params=pltpu.CompilerParams(
            dimension_semantics=("arbitrary",)),
    )(locs4, scores2, boxes, labels_f, priors_t)
    return out[0, 0]


# speedup vs baseline: 49.7476x; 1.1513x over previous
import functools
from math import sqrt

import jax
import jax.numpy as jnp
from jax import lax
from jax.experimental import pallas as pl
from jax.experimental.pallas import tpu as pltpu

_N_CLASSES = 2
_THRESHOLD = 0.5
_NEG_POS_RATIO = 3
_ALPHA = 1.0
_SMOOTHING = 0.05
_N_BISECT = 34


def _loss_kernel(locs_ref, scores_ref, boxes_ref, labels_ref, priors_ref,
                 out_ref, lab_s, tx1_s, ty1_s, tx2_s, ty2_s, *, B, P, Pp, M):
    g = pl.program_id(0)
    f32 = jnp.float32
    n_grp = B // M

    pcx = priors_ref[0:1, :]
    pcy = priors_ref[1:2, :]
    pw = priors_ref[2:3, :]
    ph = priors_ref[3:4, :]
    px1 = pcx - pw * 0.5
    py1 = pcy - ph * 0.5
    px2 = pcx + pw * 0.5
    py2 = pcy + ph * 0.5

    col1 = lax.broadcasted_iota(jnp.int32, (1, Pp), 1)
    valid_col = col1 < P

    bx = boxes_ref[0]
    x1 = bx[:, :, 0:1]
    y1 = bx[:, :, 1:2]
    x2 = bx[:, :, 2:3]
    y2 = bx[:, :, 3:4]
    lab_b = labels_ref[0]

    ix1 = jnp.maximum(x1, px1)
    iy1 = jnp.maximum(y1, py1)
    ix2 = jnp.minimum(x2, px2)
    iy2 = jnp.minimum(y2, py2)
    inter = jnp.maximum(ix2 - ix1, 0.0) * jnp.maximum(iy2 - iy1, 0.0)
    a1 = (x2 - x1) * (y2 - y1)
    a2 = (px2 - px1) * (py2 - py1)
    ov = inter / (a1 + a2 - inter + 1e-10)
    ov = jnp.where(valid_col, ov, -1.0)

    eio = lax.broadcasted_iota(jnp.int32, (M, 8, Pp), 1)
    cio = lax.broadcasted_iota(jnp.int32, (M, 8, Pp), 2)

    ofp = jnp.max(ov, axis=1, keepdims=True)
    oep = jnp.min(jnp.where(ov == ofp, eio, 8), axis=1, keepdims=True)
    ofe = jnp.max(ov, axis=2, keepdims=True)
    pfe = jnp.min(jnp.where(ov == ofe, cio, Pp), axis=2, keepdims=True)

    match = cio == pfe
    validk = ofe > 0.0
    e_last_all = jnp.max(jnp.where(match, eio, -1), axis=1, keepdims=True)
    e_last_val = jnp.max(jnp.where(match & validk, eio, -1), axis=1,
                         keepdims=True)
    force = (e_last_all >= 0) & (e_last_all == e_last_val)
    ofp = jnp.where(force, 1.0, ofp)
    oep = jnp.where(force, e_last_all, oep)

    onehot = oep == eio
    label_fp = jnp.sum(jnp.where(onehot, lab_b, 0.0), axis=1)
    tx1 = jnp.sum(jnp.where(onehot, x1, 0.0), axis=1)
    ty1 = jnp.sum(jnp.where(onehot, y1, 0.0), axis=1)
    tx2 = jnp.sum(jnp.where(onehot, x2, 0.0), axis=1)
    ty2 = jnp.sum(jnp.where(onehot, y2, 0.0), axis=1)
    label_fp = jnp.where(jnp.squeeze(ofp, 1) < _THRESHOLD - 0.1,
                         0.0, label_fp)

    lab_s[pl.ds(g * M, M), :] = label_fp
    tx1_s[pl.ds(g * M, M), :] = tx1
    ty1_s[pl.ds(g * M, M), :] = ty1
    tx2_s[pl.ds(g * M, M), :] = tx2
    ty2_s[pl.ds(g * M, M), :] = ty2

    @pl.when(g == n_grp - 1)
    def _dense():
        lab = lab_s[...]
        pos = lab > 0.0
        posf = pos.astype(f32)
        n_pos_vec = jnp.sum(posf, axis=1, keepdims=True)
        n_pos_total = jnp.sum(posf)

        gcx = locs_ref[0]
        gcy = locs_ref[1]
        gw = locs_ref[2]
        gh = locs_ref[3]
        cx = gcx * pw * 0.1 + pcx
        cy = gcy * ph * 0.1 + pcy
        w = jnp.exp(gw * 0.2) * pw
        h = jnp.exp(gh * 0.2) * ph
        dx1 = cx - w * 0.5
        dy1 = cy - h * 0.5
        dx2 = cx + w * 0.5
        dy2 = cy + h * 0.5

        ttx1 = tx1_s[...]
        tty1 = ty1_s[...]
        ttx2 = tx2_s[...]
        tty2 = ty2_s[...]

        lx1 = jnp.maximum(dx1, ttx1)
        ly1 = jnp.maximum(dy1, tty1)
        lx2 = jnp.minimum(dx2, ttx2)
        ly2 = jnp.minimum(dy2, tty2)
        inter_d = (jnp.maximum(lx2 - lx1, 0.0) * jnp.maximum(ly2 - ly1, 0.0))
        ap = jnp.maximum(dx2 - dx1, 0.0) * jnp.maximum(dy2 - dy1, 0.0)
        at = (ttx2 - ttx1) * (tty2 - tty1)
        iou = inter_d / (ap + at - inter_d + 1e-7)
        dcx = (dx1 + dx2) - (ttx1 + ttx2)
        dcy = (dy1 + dy2) - (tty1 + tty2)
        d2 = (dcx * dcx + dcy * dcy) * 0.25
        ex1 = jnp.minimum(dx1, ttx1)
        ey1 = jnp.minimum(dy1, tty1)
        ex2 = jnp.maximum(dx2, ttx2)
        ey2 = jnp.maximum(dy2, tty2)
        c2 = (ex2 - ex1) ** 2 + (ey2 - ey1) ** 2 + 1e-7
        per_box = 1.0 - iou + d2 / c2
        loc_sum = jnp.sum(jnp.where(pos, per_box, 0.0))

        s0 = scores_ref[0]
        s1 = scores_ref[1]
        m = jnp.maximum(s0, s1)
        lse = m + jnp.log(jnp.exp(s0 - m) + jnp.exp(s1 - m))
        lp0 = s0 - lse
        lp1 = s1 - lse
        lp_t = jnp.where(lab > 0.0, lp1, lp0)
        eps_i = _SMOOTHING / (_N_CLASSES - 1)
        ce = -((1.0 - _SMOOTHING) * lp_t + eps_i * (lp0 + lp1 - lp_t))
        conf_pos_sum = jnp.sum(jnp.where(pos, ce, 0.0))
        cn = jnp.where(valid_col & ~pos, ce, 0.0)

        k = jnp.minimum(_NEG_POS_RATIO * n_pos_vec, float(P))
        hi0 = jnp.max(cn, axis=1, keepdims=True) + 1.0
        lo0 = jnp.zeros_like(hi0)

        def body(_, carry):
            lo, hi = carry
            mid = 0.5 * (lo + hi)
            cnt = jnp.sum(jnp.where(cn > mid, 1.0, 0.0), axis=1,
                          keepdims=True)
            gt = cnt > k
            return jnp.where(gt, mid, lo), jnp.where(gt, hi, mid)

        _, hi = lax.fori_loop(0, _N_BISECT, body, (lo0, hi0))
        above = cn > hi
        cnt_hi = jnp.sum(jnp.where(above, 1.0, 0.0), axis=1, keepdims=True)
        sum_hi = jnp.sum(jnp.where(above, cn, 0.0), axis=1, keepdims=True)
        hard_sum = jnp.sum(sum_hi + (k - cnt_hi) * hi)

        conf_loss = (hard_sum + conf_pos_sum) / n_pos_total
        loc_loss = loc_sum / jnp.maximum(n_pos_total, 1.0)
        total = conf_loss + _ALPHA * loc_loss
        out_ref[...] = jnp.broadcast_to(total, (1, 1))


@jax.jit
def kernel(odm_locs, odm_scores, boxes, labels, priors_cxcy):
    B, P, C = odm_scores.shape
    Pp = ((P + 127) // 128) * 128
    pad = Pp - P
    M = 8
    locs4 = jnp.pad(jnp.transpose(odm_locs, (2, 0, 1)),
                    ((0, 0), (0, 0), (0, pad)))
    scores2 = jnp.pad(jnp.transpose(odm_scores, (2, 0, 1)),
                      ((0, 0), (0, 0), (0, pad)))
    priors_t = jnp.pad(priors_cxcy.T, ((0, 0), (0, pad)))
    labels_f = labels.astype(jnp.float32)[..., None]

    body = functools.partial(_loss_kernel, B=B, P=P, Pp=Pp, M=M)
    out = pl.pallas_call(
        body,
        grid=(B // M,),
        in_specs=[
            pl.BlockSpec((4, B, Pp), lambda g: (0, 0, 0)),
            pl.BlockSpec((C, B, Pp), lambda g: (0, 0, 0)),
            pl.BlockSpec((1, M, 8, 4), lambda g: (g, 0, 0, 0)),
            pl.BlockSpec((1, M, 8, 1), lambda g: (g, 0, 0, 0)),
            pl.BlockSpec((4, Pp), lambda g: (0, 0)),
        ],
        out_specs=pl.BlockSpec((1, 1), lambda g: (0, 0)),
        out_shape=jax.ShapeDtypeStruct((1, 1), jnp.float32),
        scratch_shapes=[pltpu.VMEM((B, Pp), jnp.float32) for _ in range(5)],
        compiler_params=pltpu.CompilerParams(
            dimension_semantics=("arbitrary",)),
    )(locs4, scores2, boxes.reshape(B // M, M, 8, 4),
      labels_f.reshape(B // M, M, 8, 1), priors_t)
    return out[0, 0]
